# Initial kernel scaffold; baseline (speedup 1.0000x reference)
import jax
import jax.numpy as jnp
from jax.experimental import pallas as pl


def kernel(x, edge_index, Wl1, bl1, Wr1, Wl2, bl2, Wr2, Wl3, bl3, Wr3):
    raise NotImplementedError("write your pallas kernel here")


# speedup vs baseline: 3.5936x; 3.5936x over previous
import functools

import jax
import jax.numpy as jnp
from jax import lax
from jax.experimental import pallas as pl
from jax.experimental.pallas import tpu as pltpu
from jax.experimental.pallas import tpu_sc as plsc

NC = 2
NS = 16
NW = NC * NS
C = 128
ZR = 64


def _agg_body(h_hbm, src_hbm, dst_hbm, out_hbm, srcv, dstv, rows, zbuf, acc, sem):
    k_steps = src_hbm.shape[1]
    npad = acc.shape[0]
    rpt = npad // NS
    cid = lax.axis_index("c")
    sid = lax.axis_index("s")
    wid = cid * NS + sid

    zvec = jnp.zeros((16,), jnp.float32)

    def zrow(r, carry):
        for c8 in range(8):
            zbuf[r, pl.ds(c8 * 16, 16)] = zvec
        return carry

    lax.fori_loop(0, ZR, zrow, 0)

    def zcopy(it, carry):
        pltpu.sync_copy(zbuf, acc.at[pl.ds(sid * rpt + it * ZR, ZR)])
        return carry

    lax.fori_loop(0, rpt // ZR, zcopy, 0)
    plsc.subcore_barrier()

    pltpu.sync_copy(src_hbm.at[wid], srcv)
    pltpu.sync_copy(dst_hbm.at[wid], dstv)

    def step(i, carry):
        pltpu.async_copy(h_hbm.at[srcv.at[i]], rows, sem).wait()
        pltpu.sync_copy(rows, acc.at[dstv.at[i]], add=True)
        return carry

    lax.fori_loop(0, k_steps, step, 0)
    plsc.subcore_barrier()

    pltpu.sync_copy(acc.at[pl.ds(sid * rpt, rpt)],
                    out_hbm.at[cid, pl.ds(sid * rpt, rpt)])


def _combine_body(relu, p0, p1, d0, d1, h, wl, bl, wr, o):
    deg = jnp.maximum(d0[:, 0:1] + d1[:, 0:1], 1.0)
    mean = (p0[...] + p1[...]) / deg
    r = (jnp.dot(mean, wl[...], preferred_element_type=jnp.float32)
         + jnp.dot(h[...], wr[...], preferred_element_type=jnp.float32)
         + bl[...])
    o[...] = jnp.maximum(r, 0.0) if relu else r


@functools.lru_cache(maxsize=None)
def _make_agg(npad, k_steps, d):
    mesh = plsc.VectorSubcoreMesh(core_axis_name="c", subcore_axis_name="s")
    return pl.kernel(
        _agg_body,
        mesh=mesh,
        out_type=jax.ShapeDtypeStruct((NC, npad, d), jnp.float32),
        scratch_types=[
            pltpu.VMEM((k_steps, C), jnp.int32),
            pltpu.VMEM((k_steps, C), jnp.int32),
            pltpu.VMEM((C, d), jnp.float32),
            pltpu.VMEM((ZR, d), jnp.float32),
            pltpu.VMEM_SHARED((npad, d), jnp.float32),
            pltpu.SemaphoreType.DMA,
        ],
    )


@functools.lru_cache(maxsize=None)
def _make_combine(npad, d, relu, blk=256):
    bs_rows = pl.BlockSpec((blk, d), lambda i: (i, 0))
    bs_deg = pl.BlockSpec((blk, d), lambda i: (i, 0))
    bs_w = pl.BlockSpec((d, d), lambda i: (0, 0))
    bs_b = pl.BlockSpec((1, d), lambda i: (0, 0))
    return pl.pallas_call(
        functools.partial(_combine_body, relu),
        grid=(npad // blk,),
        in_specs=[bs_rows, bs_rows, bs_deg, bs_deg, bs_rows, bs_w, bs_b, bs_w],
        out_specs=bs_rows,
        out_shape=jax.ShapeDtypeStruct((npad, d), jnp.float32),
    )


def kernel(x, edge_index, Wl1, bl1, Wr1, Wl2, bl2, Wr2, Wl3, bl3, Wr3):
    n, d = x.shape
    e = edge_index.shape[1]

    npad = -(-n // (NS * ZR)) * (NS * ZR)
    k_steps = -(-e // (NW * C))
    epad = NW * k_steps * C

    src = jnp.concatenate(
        [edge_index[0], jnp.zeros((epad - e,), jnp.int32)]).reshape(NW, k_steps, C)
    dst = jnp.concatenate(
        [edge_index[1], jnp.full((epad - e,), npad - 1, jnp.int32)]).reshape(NW, k_steps, C)

    xp = jnp.pad(x, ((0, npad - n), (0, 0)))

    agg = _make_agg(npad, k_steps, d)
    deg = agg(jnp.ones((npad, d), jnp.float32), src, dst)

    h = xp
    for wl, bl, wr, relu in ((Wl1, bl1, Wr1, True),
                             (Wl2, bl2, Wr2, True),
                             (Wl3, bl3, Wr3, False)):
        parts = agg(h, src, dst)
        h = _make_combine(npad, d, relu)(
            parts[0], parts[1], deg[0], deg[1], h, wl, bl.reshape(1, d), wr)
    return h[:n]


# speedup vs baseline: 4.6051x; 1.2815x over previous
import functools

import jax
import jax.numpy as jnp
from jax import lax
from jax.experimental import pallas as pl
from jax.experimental.pallas import tpu as pltpu
from jax.experimental.pallas import tpu_sc as plsc

NC = 2
NS = 16
NW = NC * NS
C = 80
ZR = 16


def _agg_body(h_hbm, src_hbm, dst_hbm, out_hbm, srcv, dstv, rows_a,
              zbuf, acc, sem_a):
    k_steps = src_hbm.shape[1]
    npad = acc.shape[0]
    rpt = npad // NS
    cid = lax.axis_index("c")
    sid = lax.axis_index("s")
    wid = cid * NS + sid

    zvec = jnp.zeros((16,), jnp.float32)

    def zrow(r, carry):
        for c8 in range(8):
            zbuf[r, pl.ds(c8 * 16, 16)] = zvec
        return carry

    lax.fori_loop(0, ZR, zrow, 0)

    def zcopy(it, carry):
        pltpu.sync_copy(zbuf, acc.at[pl.ds(sid * rpt + it * ZR, ZR)])
        return carry

    lax.fori_loop(0, rpt // ZR, zcopy, 0)
    plsc.subcore_barrier()

    pltpu.sync_copy(src_hbm.at[wid], srcv)
    pltpu.sync_copy(dst_hbm.at[wid], dstv)

    def step(i, carry):
        pltpu.async_copy(h_hbm.at[srcv.at[i]], rows_a, sem_a)
        pltpu.make_async_copy(h_hbm.at[srcv.at[i]], rows_a, sem_a).wait()
        pltpu.sync_copy(rows_a, acc.at[dstv.at[i]], add=True)
        return carry

    lax.fori_loop(0, k_steps, step, 0)
    plsc.subcore_barrier()

    pltpu.sync_copy(acc.at[pl.ds(sid * rpt, rpt)],
                    out_hbm.at[cid, pl.ds(sid * rpt, rpt)])


def _deg_body(dst_hbm, out_hbm, dstv, ones_buf, zbuf, acc):
    k_steps = dst_hbm.shape[1]
    npad = acc.shape[0]
    d = acc.shape[1]
    rpt = npad // NS
    cid = lax.axis_index("c")
    sid = lax.axis_index("s")
    wid = cid * NS + sid

    zvec = jnp.zeros((16,), jnp.float32)
    ovec = jnp.ones((16,), jnp.float32)

    def zrow(r, carry):
        for c8 in range(d // 16):
            zbuf[r, pl.ds(c8 * 16, 16)] = zvec
        return carry

    lax.fori_loop(0, ZR, zrow, 0)

    def orow(r, carry):
        for c8 in range(d // 16):
            ones_buf[r, pl.ds(c8 * 16, 16)] = ovec
        return carry

    lax.fori_loop(0, C, orow, 0)

    def zcopy(it, carry):
        pltpu.sync_copy(zbuf, acc.at[pl.ds(sid * rpt + it * ZR, ZR)])
        return carry

    lax.fori_loop(0, rpt // ZR, zcopy, 0)
    plsc.subcore_barrier()

    pltpu.sync_copy(dst_hbm.at[wid], dstv)

    def step(i, carry):
        pltpu.sync_copy(ones_buf, acc.at[dstv.at[i]], add=True)
        return carry

    lax.fori_loop(0, k_steps, step, 0)
    plsc.subcore_barrier()

    pltpu.sync_copy(acc.at[pl.ds(sid * rpt, rpt)],
                    out_hbm.at[cid, pl.ds(sid * rpt, rpt)])


def _combine_body(relu, p0, p1, d0, d1, h, wl, bl, wr, o):
    deg = jnp.maximum(d0[:, 0:1] + d1[:, 0:1], 1.0)
    mean = (p0[...] + p1[...]) / deg
    r = (jnp.dot(mean, wl[...], preferred_element_type=jnp.float32)
         + jnp.dot(h[...], wr[...], preferred_element_type=jnp.float32)
         + bl[...])
    o[...] = jnp.maximum(r, 0.0) if relu else r


@functools.lru_cache(maxsize=None)
def _make_agg(npad, k_steps, d):
    mesh = plsc.VectorSubcoreMesh(core_axis_name="c", subcore_axis_name="s")
    return pl.kernel(
        _agg_body,
        mesh=mesh,
        out_type=jax.ShapeDtypeStruct((NC, npad, d), jnp.float32),
        scratch_types=[
            pltpu.VMEM((k_steps, C), jnp.int32),
            pltpu.VMEM((k_steps, C), jnp.int32),
            pltpu.VMEM((C, d), jnp.float32),
            pltpu.VMEM((ZR, d), jnp.float32),
            pltpu.VMEM_SHARED((npad, d), jnp.float32),
            pltpu.SemaphoreType.DMA,
        ],
    )


@functools.lru_cache(maxsize=None)
def _make_deg(npad, k_steps, d):
    mesh = plsc.VectorSubcoreMesh(core_axis_name="c", subcore_axis_name="s")
    return pl.kernel(
        _deg_body,
        mesh=mesh,
        out_type=jax.ShapeDtypeStruct((NC, npad, d), jnp.float32),
        scratch_types=[
            pltpu.VMEM((k_steps, C), jnp.int32),
            pltpu.VMEM((C, d), jnp.float32),
            pltpu.VMEM((ZR, d), jnp.float32),
            pltpu.VMEM_SHARED((npad, d), jnp.float32),
        ],
    )


@functools.lru_cache(maxsize=None)
def _make_combine(npad, d, relu, blk=256):
    bs_rows = pl.BlockSpec((blk, d), lambda i: (i, 0))
    bs_deg = pl.BlockSpec((blk, d), lambda i: (i, 0))
    bs_w = pl.BlockSpec((d, d), lambda i: (0, 0))
    bs_b = pl.BlockSpec((1, d), lambda i: (0, 0))
    return pl.pallas_call(
        functools.partial(_combine_body, relu),
        grid=(npad // blk,),
        in_specs=[bs_rows, bs_rows, bs_deg, bs_deg, bs_rows, bs_w, bs_b, bs_w],
        out_specs=bs_rows,
        out_shape=jax.ShapeDtypeStruct((npad, d), jnp.float32),
    )


def kernel(x, edge_index, Wl1, bl1, Wr1, Wl2, bl2, Wr2, Wl3, bl3, Wr3):
    n, d = x.shape
    e = edge_index.shape[1]

    npad = -(-n // (NS * ZR)) * (NS * ZR)
    k_steps = -(-e // (NW * C))
    k_steps += k_steps % 2
    epad = NW * k_steps * C

    src = jnp.concatenate(
        [edge_index[0], jnp.zeros((epad - e,), jnp.int32)]).reshape(NW, k_steps, C)
    dst = jnp.concatenate(
        [edge_index[1], jnp.full((epad - e,), npad - 1, jnp.int32)]).reshape(NW, k_steps, C)

    xp = jnp.pad(x, ((0, npad - n), (0, 0)))

    agg = _make_agg(npad, k_steps, d)
    deg = _make_deg(npad, k_steps, d)(dst)
    xp, deg = lax.optimization_barrier((xp, deg))

    h = xp
    for wl, bl, wr, relu in ((Wl1, bl1, Wr1, True),
                             (Wl2, bl2, Wr2, True),
                             (Wl3, bl3, Wr3, False)):
        parts = agg(h, src, dst)
        h = _make_combine(npad, d, relu)(
            parts[0], parts[1], deg[0], deg[1], h, wl, bl.reshape(1, d), wr)
    return h[:n]
